# baseline (speedup 1.0000x reference)
import dataclasses
import functools

import jax
import jax.numpy as jnp
from jax import lax
from jax.experimental import pallas as pl
from jax.experimental.pallas import tpu as pltpu
from jax.experimental.pallas import tpu_sc as plsc

_N_C, _N_T, _D = 10000, 1000, 128
_E_C, _E_T = 320000, 16000
_L = 2
_EPS = 1e-5

_NCORE, _NSUB, _NW = 2, 16, 32
_N_C_PAD = 10112
_N_T_PAD = 1024
_E_C_PAD = 327680
_E_T_PAD = 16384
_CG_PAD = 10240

_MESH = plsc.VectorSubcoreMesh(core_axis_name="c", subcore_axis_name="s")
_F32 = jnp.float32
_I32 = jnp.int32

_SC_PARAMS = pltpu.CompilerParams()
if "needs_layout_passes" in pltpu.CompilerParams.__dataclass_fields__:
    _SC_PARAMS = dataclasses.replace(_SC_PARAMS, needs_layout_passes=False)


def _make_edge_agg(n_pad, n_batches, with_deg):
    rows_ps = n_pad // _NSUB

    outs = [jax.ShapeDtypeStruct((_NCORE, n_pad, _D), _F32)]
    if with_deg:
        outs.append(jax.ShapeDtypeStruct((_NW, n_pad), _F32))
    scratch = [
        pltpu.VMEM((n_batches, 128), _I32),
        pltpu.VMEM((n_batches, 128), _I32),
        pltpu.VMEM((128, _D), _F32),
        pltpu.VMEM_SHARED((n_pad, _D), _F32),
        pltpu.SemaphoreType.DMA,
    ]
    if with_deg:
        scratch.append(pltpu.VMEM((n_pad,), _F32))

    def body(h_hbm, src_hbm, dst_hbm, zeros_hbm, *refs):
        if with_deg:
            (out_hbm, deg_hbm, idx_s, idx_d, rows_a, agg_sh, sem_a,
             deg_v) = refs
        else:
            (out_hbm, idx_s, idx_d, rows_a, agg_sh, sem_a) = refs
        c = lax.axis_index("c")
        s = lax.axis_index("s")
        wid = c * _NSUB + s
        r0 = s * rows_ps
        pltpu.sync_copy(zeros_hbm.at[pl.ds(r0, rows_ps)],
                        agg_sh.at[pl.ds(r0, rows_ps)])
        if with_deg:
            @pl.loop(0, n_pad, step=16)
            def _(i):
                deg_v[pl.ds(i, 16)] = jnp.zeros((16,), _F32)
        pltpu.sync_copy(src_hbm.at[wid], idx_s)
        pltpu.sync_copy(dst_hbm.at[wid], idx_d)
        plsc.subcore_barrier()

        def deg_ops(bi):
            @pl.loop(0, 128, step=16)
            def _(k):
                plsc.addupdate_scatter(
                    deg_v, [idx_d[bi, pl.ds(k, 16)]],
                    jnp.full((16,), 1.0, _F32))

        @pl.loop(0, n_batches)
        def _(i):
            pltpu.async_copy(h_hbm.at[idx_s.at[i]], rows_a, sem_a).wait()
            if with_deg:
                deg_ops(i)
            pltpu.sync_copy(rows_a, agg_sh.at[idx_d.at[i]], add=True)

        plsc.subcore_barrier()
        pltpu.sync_copy(agg_sh.at[pl.ds(r0, rows_ps)],
                        out_hbm.at[c, pl.ds(r0, rows_ps)])
        if with_deg:
            pltpu.sync_copy(deg_v, deg_hbm.at[wid])

    return pl.kernel(body, out_type=tuple(outs) if with_deg else outs[0],
                     mesh=_MESH, scratch_types=scratch,
                     compiler_params=_SC_PARAMS)


def _make_pool():
    rows_ps = _N_T_PAD // _NSUB
    per_w = _CG_PAD // _NW
    t_batches = _E_T_PAD // (_NW * 128)

    outs = (jax.ShapeDtypeStruct((_NCORE, _N_T_PAD, _D), _F32),
            jax.ShapeDtypeStruct((_NW, _N_T_PAD), _F32))
    scratch = [
        pltpu.VMEM((5, 64), _I32),
        pltpu.VMEM((t_batches, 128), _I32),
        pltpu.VMEM((64, _D), _F32),
        pltpu.VMEM_SHARED((_N_T_PAD, _D), _F32),
        pltpu.SemaphoreType.DMA,
        pltpu.VMEM((_N_T_PAD,), _F32),
    ]

    def body(cg_hbm, cols_hbm, tdst_hbm, zeros_hbm, out_hbm, tdeg_hbm,
             idx_p, idx_t, rows, pool_sh, sem, deg_v):
        c = lax.axis_index("c")
        s = lax.axis_index("s")
        wid = c * _NSUB + s
        r0 = s * rows_ps
        pltpu.sync_copy(zeros_hbm.at[pl.ds(r0, rows_ps)],
                        pool_sh.at[pl.ds(r0, rows_ps)])

        @pl.loop(0, _N_T_PAD, step=16)
        def _(i):
            deg_v[pl.ds(i, 16)] = jnp.zeros((16,), _F32)

        pltpu.sync_copy(cols_hbm.at[wid], idx_p)
        pltpu.sync_copy(tdst_hbm.at[wid], idx_t)
        plsc.subcore_barrier()

        base = wid * per_w

        @pl.loop(0, 5)
        def _(bi):
            pltpu.sync_copy(cg_hbm.at[pl.ds(base + bi * 64, 64)], rows)
            pltpu.sync_copy(rows, pool_sh.at[idx_p.at[bi]], add=True)

        @pl.loop(0, t_batches)
        def _(bi):
            @pl.loop(0, 128, step=16)
            def _(k):
                plsc.addupdate_scatter(
                    deg_v, [idx_t[bi, pl.ds(k, 16)]],
                    jnp.full((16,), 1.0, _F32))

        plsc.subcore_barrier()
        pltpu.sync_copy(pool_sh.at[pl.ds(r0, rows_ps)],
                        out_hbm.at[c, pl.ds(r0, rows_ps)])
        pltpu.sync_copy(deg_v, tdeg_hbm.at[wid])

    return pl.kernel(body, out_type=outs, mesh=_MESH, scratch_types=scratch,
                     compiler_params=_SC_PARAMS)


_cell_agg_deg = _make_edge_agg(_N_C_PAD, _E_C_PAD // (_NW * 128), True)
_cell_agg = _make_edge_agg(_N_C_PAD, _E_C_PAD // (_NW * 128), False)
_tissue_agg = _make_edge_agg(_N_T_PAD, _E_T_PAD // (_NW * 128), False)
_pool_scatter = _make_pool()


def _dot(x, w):
    return jax.lax.dot_general(x, w, (((1,), (0,)), ((), ())),
                               preferred_element_type=_F32)


def _norm2_relu(y, prm):
    g1, b1, g2, b2 = prm[1][None], prm[2][None], prm[3][None], prm[4][None]
    mu = jnp.mean(y, axis=0, keepdims=True)
    var = jnp.mean((y - mu) ** 2, axis=0, keepdims=True)
    s1 = jnp.sqrt(var + _EPS)
    v2 = g1 * g1 * var / (var + _EPS)
    s2 = jnp.sqrt(v2 + _EPS)
    return jnp.maximum((y - mu) * (g1 * g2 / (s1 * s2)) + b2, 0.0)


def _combine(p_ref, degp_ref, prm_ref, n):
    y = p_ref[0, :n, :] + p_ref[1, :n, :]
    deg = jnp.maximum(jnp.sum(degp_ref[...], axis=0)[:n], 1.0)
    y = y / deg[:, None] + prm_ref[0][None]
    return _norm2_relu(y, prm_ref)


def _mm_body(x_ref, w_ref, o_ref):
    o_ref[...] = _dot(x_ref[...], w_ref[...])


def _make_post(n):
    def body(p_ref, degp_ref, w_ref, prm_ref, o_ref):
        o_ref[...] = _dot(_combine(p_ref, degp_ref, prm_ref, n), w_ref[...])
    return body


def _make_final(n, out_rows):
    def body(p_ref, degp_ref, prm_ref, o_ref):
        x = _combine(p_ref, degp_ref, prm_ref, n)
        if out_rows > n:
            o_ref[:n] = x
            o_ref[n:] = jnp.zeros((out_rows - n, _D), _F32)
        else:
            o_ref[...] = x
    return body


def _argmax_body(a_ref, o_ref):
    x = a_ref[...]
    m = jnp.max(x, axis=1, keepdims=True)
    ii = lax.broadcasted_iota(_I32, x.shape, 1)
    idx = jnp.min(jnp.where(x == m, ii, jnp.int32(2**30)), axis=1)
    o_ref[...] = idx[:, None].astype(_I32)


def _tg_body(p_ref, tf_ref, w_ref, o_ref):
    tg = p_ref[0, :_N_T, :] + p_ref[1, :_N_T, :] + tf_ref[...]
    o_ref[...] = _dot(tg, w_ref[...])


def _sds(shape, dtype=_F32):
    return jax.ShapeDtypeStruct(shape, dtype)


def kernel(cell_feat, tissue_feat, assignment_mat, Ws, bs, gn_g, gn_b,
           bn_g, bn_b, cell_edge_index, tissue_edge_index):
    nb_c = _E_C_PAD // (_NW * 128)
    nb_t = _E_T_PAD // (_NW * 128)
    zeros_big = jnp.zeros((_N_C_PAD, _D), _F32)

    csrc = jnp.concatenate(
        [cell_edge_index[0], jnp.zeros((_E_C_PAD - _E_C,), _I32)]
    ).reshape(_NW, nb_c, 128)
    cdst = jnp.concatenate(
        [cell_edge_index[1], jnp.full((_E_C_PAD - _E_C,), _N_C, _I32)]
    ).reshape(_NW, nb_c, 128)
    tsrc = jnp.concatenate(
        [tissue_edge_index[0], jnp.zeros((_E_T_PAD - _E_T,), _I32)]
    ).reshape(_NW, nb_t, 128)
    tdst = jnp.concatenate(
        [tissue_edge_index[1], jnp.full((_E_T_PAD - _E_T,), _N_T, _I32)]
    ).reshape(_NW, nb_t, 128)

    prm = [jnp.stack([bs[i], gn_g[i], gn_b[i], bn_g[i], bn_b[i]])
           for i in range(_L)]

    cols2d = pl.pallas_call(
        _argmax_body,
        grid=(5,),
        in_specs=[pl.BlockSpec((2000, _N_T), lambda i: (i, 0))],
        out_specs=pl.BlockSpec((2000, 1), lambda i: (i, 0)),
        out_shape=_sds((_N_C, 1), _I32),
    )(assignment_mat)
    cols = jnp.concatenate(
        [cols2d[:, 0], jnp.zeros((_CG_PAD - _N_C,), _I32)]
    ).reshape(_NW, 5, 64)

    h1 = pl.pallas_call(_mm_body, out_shape=_sds((_N_C, _D)))(
        cell_feat, Ws[0])
    agg1, degp = _cell_agg_deg(h1, csrc, cdst, zeros_big)
    h2 = pl.pallas_call(_make_post(_N_C), out_shape=_sds((_N_C, _D)))(
        agg1, degp, Ws[1], prm[0])
    agg2 = _cell_agg(h2, csrc, cdst, zeros_big)
    cg = pl.pallas_call(_make_final(_N_C, _CG_PAD),
                        out_shape=_sds((_CG_PAD, _D)))(agg2, degp, prm[1])

    poolp, tdegp = _pool_scatter(cg, cols, tdst, zeros_big)
    ht1 = pl.pallas_call(_tg_body, out_shape=_sds((_N_T, _D)))(
        poolp, tissue_feat, Ws[0])

    tagg1 = _tissue_agg(ht1, tsrc, tdst, zeros_big)
    ht2 = pl.pallas_call(_make_post(_N_T), out_shape=_sds((_N_T, _D)))(
        tagg1, tdegp, Ws[1], prm[0])
    tagg2 = _tissue_agg(ht2, tsrc, tdst, zeros_big)
    out = pl.pallas_call(_make_final(_N_T, _N_T),
                         out_shape=_sds((_N_T, _D)))(tagg2, tdegp, prm[1])
    return out


# speedup vs baseline: 1.0070x; 1.0070x over previous
import dataclasses
import functools

import jax
import jax.numpy as jnp
from jax import lax
from jax.experimental import pallas as pl
from jax.experimental.pallas import tpu as pltpu
from jax.experimental.pallas import tpu_sc as plsc

_N_C, _N_T, _D = 10000, 1000, 128
_E_C, _E_T = 320000, 16000
_L = 2
_EPS = 1e-5

_NCORE, _NSUB, _NW = 2, 16, 32
_N_C_PAD = 10112
_N_T_PAD = 1024
_E_C_PAD = 327680
_E_T_PAD = 16384
_CG_PAD = 10240

_MESH = plsc.VectorSubcoreMesh(core_axis_name="c", subcore_axis_name="s")
_F32 = jnp.float32
_I32 = jnp.int32

_SC_PARAMS = pltpu.CompilerParams()
if "needs_layout_passes" in pltpu.CompilerParams.__dataclass_fields__:
    _SC_PARAMS = dataclasses.replace(_SC_PARAMS, needs_layout_passes=False)


def _make_edge_agg(n_pad, n_batches, with_deg):
    rows_ps = n_pad // _NSUB

    outs = [jax.ShapeDtypeStruct((_NCORE, n_pad, _D), _F32)]
    if with_deg:
        outs.append(jax.ShapeDtypeStruct((_NW, n_pad), _F32))
    scratch = [
        pltpu.VMEM((n_batches, 128), _I32),
        pltpu.VMEM((n_batches, 128), _I32),
        pltpu.VMEM((128, _D), _F32),
        pltpu.VMEM_SHARED((n_pad, _D), _F32),
        pltpu.SemaphoreType.DMA,
        pltpu.SemaphoreType.DMA,
    ]
    if with_deg:
        scratch.append(pltpu.VMEM((n_pad,), _F32))

    def body(h_hbm, src_hbm, dst_hbm, zeros_hbm, *refs):
        if with_deg:
            (out_hbm, deg_hbm, idx_s, idx_d, rows_a, agg_sh, sem_g,
             sem_s, deg_v) = refs
        else:
            (out_hbm, idx_s, idx_d, rows_a, agg_sh, sem_g, sem_s) = refs
        c = lax.axis_index("c")
        s = lax.axis_index("s")
        wid = c * _NSUB + s
        r0 = s * rows_ps
        pltpu.sync_copy(zeros_hbm.at[pl.ds(r0, rows_ps)],
                        agg_sh.at[pl.ds(r0, rows_ps)])
        if with_deg:
            @pl.loop(0, n_pad, step=16)
            def _(i):
                deg_v[pl.ds(i, 16)] = jnp.zeros((16,), _F32)
        pltpu.sync_copy(src_hbm.at[wid], idx_s)
        pltpu.sync_copy(dst_hbm.at[wid], idx_d)
        plsc.subcore_barrier()

        @pl.loop(0, n_batches)
        def _(i):
            pltpu.async_copy(h_hbm.at[idx_s.at[i]], rows_a, sem_g).wait()
            if with_deg:
                scat = pltpu.async_copy(rows_a, agg_sh.at[idx_d.at[i]],
                                        sem_s, add=True)

                @pl.loop(0, 128, step=16)
                def _(k):
                    plsc.addupdate_scatter(
                        deg_v, [idx_d[i, pl.ds(k, 16)]],
                        jnp.full((16,), 1.0, _F32))

                scat.wait()
            else:
                pltpu.sync_copy(rows_a, agg_sh.at[idx_d.at[i]], add=True)

        plsc.subcore_barrier()
        pltpu.sync_copy(agg_sh.at[pl.ds(r0, rows_ps)],
                        out_hbm.at[c, pl.ds(r0, rows_ps)])
        if with_deg:
            pltpu.sync_copy(deg_v, deg_hbm.at[wid])

    return pl.kernel(body, out_type=tuple(outs) if with_deg else outs[0],
                     mesh=_MESH, scratch_types=scratch,
                     compiler_params=_SC_PARAMS)


def _make_pool():
    rows_ps = _N_T_PAD // _NSUB
    per_w = _CG_PAD // _NW
    t_batches = _E_T_PAD // (_NW * 128)

    outs = (jax.ShapeDtypeStruct((_NCORE, _N_T_PAD, _D), _F32),
            jax.ShapeDtypeStruct((_NW, _N_T_PAD), _F32))
    scratch = [
        pltpu.VMEM((5, 64), _I32),
        pltpu.VMEM((t_batches, 128), _I32),
        pltpu.VMEM((64, _D), _F32),
        pltpu.VMEM_SHARED((_N_T_PAD, _D), _F32),
        pltpu.SemaphoreType.DMA,
        pltpu.VMEM((_N_T_PAD,), _F32),
    ]

    def body(cg_hbm, cols_hbm, tdst_hbm, zeros_hbm, out_hbm, tdeg_hbm,
             idx_p, idx_t, rows, pool_sh, sem, deg_v):
        c = lax.axis_index("c")
        s = lax.axis_index("s")
        wid = c * _NSUB + s
        r0 = s * rows_ps
        pltpu.sync_copy(zeros_hbm.at[pl.ds(r0, rows_ps)],
                        pool_sh.at[pl.ds(r0, rows_ps)])

        @pl.loop(0, _N_T_PAD, step=16)
        def _(i):
            deg_v[pl.ds(i, 16)] = jnp.zeros((16,), _F32)

        pltpu.sync_copy(cols_hbm.at[wid], idx_p)
        pltpu.sync_copy(tdst_hbm.at[wid], idx_t)
        plsc.subcore_barrier()

        base = wid * per_w

        @pl.loop(0, 5)
        def _(bi):
            pltpu.sync_copy(cg_hbm.at[pl.ds(base + bi * 64, 64)], rows)
            pltpu.sync_copy(rows, pool_sh.at[idx_p.at[bi]], add=True)

        @pl.loop(0, t_batches)
        def _(bi):
            @pl.loop(0, 128, step=16)
            def _(k):
                plsc.addupdate_scatter(
                    deg_v, [idx_t[bi, pl.ds(k, 16)]],
                    jnp.full((16,), 1.0, _F32))

        plsc.subcore_barrier()
        pltpu.sync_copy(pool_sh.at[pl.ds(r0, rows_ps)],
                        out_hbm.at[c, pl.ds(r0, rows_ps)])
        pltpu.sync_copy(deg_v, tdeg_hbm.at[wid])

    return pl.kernel(body, out_type=outs, mesh=_MESH, scratch_types=scratch,
                     compiler_params=_SC_PARAMS)


_cell_agg_deg = _make_edge_agg(_N_C_PAD, _E_C_PAD // (_NW * 128), True)
_cell_agg = _make_edge_agg(_N_C_PAD, _E_C_PAD // (_NW * 128), False)
_tissue_agg = _make_edge_agg(_N_T_PAD, _E_T_PAD // (_NW * 128), False)
_pool_scatter = _make_pool()


def _dot(x, w):
    return jax.lax.dot_general(x, w, (((1,), (0,)), ((), ())),
                               preferred_element_type=_F32)


def _norm2_relu(y, prm):
    g1, b1, g2, b2 = prm[1][None], prm[2][None], prm[3][None], prm[4][None]
    mu = jnp.mean(y, axis=0, keepdims=True)
    var = jnp.mean((y - mu) ** 2, axis=0, keepdims=True)
    s1 = jnp.sqrt(var + _EPS)
    v2 = g1 * g1 * var / (var + _EPS)
    s2 = jnp.sqrt(v2 + _EPS)
    return jnp.maximum((y - mu) * (g1 * g2 / (s1 * s2)) + b2, 0.0)


def _combine(p_ref, degp_ref, prm_ref, n):
    y = p_ref[0, :n, :] + p_ref[1, :n, :]
    deg = jnp.maximum(jnp.sum(degp_ref[...], axis=0)[:n], 1.0)
    y = y / deg[:, None] + prm_ref[0][None]
    return _norm2_relu(y, prm_ref)


def _mm_body(x_ref, w_ref, o_ref):
    o_ref[...] = _dot(x_ref[...], w_ref[...])


def _make_post(n):
    def body(p_ref, degp_ref, w_ref, prm_ref, o_ref):
        o_ref[...] = _dot(_combine(p_ref, degp_ref, prm_ref, n), w_ref[...])
    return body


def _make_final(n, out_rows):
    def body(p_ref, degp_ref, prm_ref, o_ref):
        x = _combine(p_ref, degp_ref, prm_ref, n)
        if out_rows > n:
            o_ref[:n] = x
            o_ref[n:] = jnp.zeros((out_rows - n, _D), _F32)
        else:
            o_ref[...] = x
    return body


def _argmax_body(a_ref, o_ref):
    x = a_ref[...]
    m = jnp.max(x, axis=1, keepdims=True)
    ii = lax.broadcasted_iota(_I32, x.shape, 1)
    idx = jnp.min(jnp.where(x == m, ii, jnp.int32(2**30)), axis=1)
    o_ref[...] = idx[:, None].astype(_I32)


def _tg_body(p_ref, tf_ref, w_ref, o_ref):
    tg = p_ref[0, :_N_T, :] + p_ref[1, :_N_T, :] + tf_ref[...]
    o_ref[...] = _dot(tg, w_ref[...])


def _sds(shape, dtype=_F32):
    return jax.ShapeDtypeStruct(shape, dtype)


def kernel(cell_feat, tissue_feat, assignment_mat, Ws, bs, gn_g, gn_b,
           bn_g, bn_b, cell_edge_index, tissue_edge_index):
    nb_c = _E_C_PAD // (_NW * 128)
    nb_t = _E_T_PAD // (_NW * 128)
    zeros_big = jnp.zeros((_N_C_PAD, _D), _F32)

    csrc = jnp.concatenate(
        [cell_edge_index[0], jnp.zeros((_E_C_PAD - _E_C,), _I32)]
    ).reshape(_NW, nb_c, 128)
    cdst = jnp.concatenate(
        [cell_edge_index[1], jnp.full((_E_C_PAD - _E_C,), _N_C, _I32)]
    ).reshape(_NW, nb_c, 128)
    tsrc = jnp.concatenate(
        [tissue_edge_index[0], jnp.zeros((_E_T_PAD - _E_T,), _I32)]
    ).reshape(_NW, nb_t, 128)
    tdst = jnp.concatenate(
        [tissue_edge_index[1], jnp.full((_E_T_PAD - _E_T,), _N_T, _I32)]
    ).reshape(_NW, nb_t, 128)

    prm = [jnp.stack([bs[i], gn_g[i], gn_b[i], bn_g[i], bn_b[i]])
           for i in range(_L)]

    cols2d = pl.pallas_call(
        _argmax_body,
        grid=(5,),
        in_specs=[pl.BlockSpec((2000, _N_T), lambda i: (i, 0))],
        out_specs=pl.BlockSpec((2000, 1), lambda i: (i, 0)),
        out_shape=_sds((_N_C, 1), _I32),
    )(assignment_mat)
    cols = jnp.concatenate(
        [cols2d[:, 0], jnp.zeros((_CG_PAD - _N_C,), _I32)]
    ).reshape(_NW, 5, 64)

    h1 = pl.pallas_call(_mm_body, out_shape=_sds((_N_C, _D)))(
        cell_feat, Ws[0])
    agg1, degp = _cell_agg_deg(h1, csrc, cdst, zeros_big)
    h2 = pl.pallas_call(_make_post(_N_C), out_shape=_sds((_N_C, _D)))(
        agg1, degp, Ws[1], prm[0])
    agg2 = _cell_agg(h2, csrc, cdst, zeros_big)
    cg = pl.pallas_call(_make_final(_N_C, _CG_PAD),
                        out_shape=_sds((_CG_PAD, _D)))(agg2, degp, prm[1])

    poolp, tdegp = _pool_scatter(cg, cols, tdst, zeros_big)
    ht1 = pl.pallas_call(_tg_body, out_shape=_sds((_N_T, _D)))(
        poolp, tissue_feat, Ws[0])

    tagg1 = _tissue_agg(ht1, tsrc, tdst, zeros_big)
    ht2 = pl.pallas_call(_make_post(_N_T), out_shape=_sds((_N_T, _D)))(
        tagg1, tdegp, Ws[1], prm[0])
    tagg2 = _tissue_agg(ht2, tsrc, tdst, zeros_big)
    out = pl.pallas_call(_make_final(_N_T, _N_T),
                         out_shape=_sds((_N_T, _D)))(tagg2, tdegp, prm[1])
    return out


# speedup vs baseline: 1.5977x; 1.5866x over previous
import dataclasses
import functools

import jax
import jax.numpy as jnp
from jax import lax
from jax.experimental import pallas as pl
from jax.experimental.pallas import tpu as pltpu
from jax.experimental.pallas import tpu_sc as plsc

_N_C, _N_T, _D = 10000, 1000, 128
_E_C, _E_T = 320000, 16000
_L = 2
_EPS = 1e-5

_NCORE, _NSUB, _NW = 2, 16, 32
_N_C_PAD = 10112
_N_T_PAD = 1024
_E_C_PAD = 323584
_E_T_PAD = 16384
_CG_PAD = 10240

_MESH = plsc.VectorSubcoreMesh(core_axis_name="c", subcore_axis_name="s")
_F32 = jnp.float32
_I32 = jnp.int32

_SC_PARAMS = pltpu.CompilerParams()
if "needs_layout_passes" in pltpu.CompilerParams.__dataclass_fields__:
    _SC_PARAMS = dataclasses.replace(_SC_PARAMS, needs_layout_passes=False)


def _make_edge_agg(n_pad, n_batches, with_deg):
    rows_ps = n_pad // _NSUB

    outs = [jax.ShapeDtypeStruct((_NCORE, n_pad, _D), _F32)]
    if with_deg:
        outs.append(jax.ShapeDtypeStruct((_NW, n_pad), _F32))
    scratch = [
        pltpu.VMEM((n_batches, 128), _I32),
        pltpu.VMEM((n_batches, 128), _I32),
        pltpu.VMEM((128, _D), _F32),
        pltpu.VMEM_SHARED((n_pad, _D), _F32),
        pltpu.SemaphoreType.DMA,
    ]
    if with_deg:
        scratch.append(pltpu.VMEM((n_pad,), _F32))

    def body(h_hbm, src_hbm, dst_hbm, zeros_hbm, *refs):
        if with_deg:
            (out_hbm, deg_hbm, idx_s, idx_d, rows_a, agg_sh, sem_g,
             deg_v) = refs
        else:
            (out_hbm, idx_s, idx_d, rows_a, agg_sh, sem_g) = refs
        c = lax.axis_index("c")
        s = lax.axis_index("s")
        wid = c * _NSUB + s
        r0 = s * rows_ps
        pltpu.sync_copy(zeros_hbm.at[pl.ds(r0, rows_ps)],
                        agg_sh.at[pl.ds(r0, rows_ps)])
        if with_deg:
            @pl.loop(0, n_pad, step=16)
            def _(i):
                deg_v[pl.ds(i, 16)] = jnp.zeros((16,), _F32)
        pltpu.sync_copy(src_hbm.at[wid], idx_s)
        pltpu.sync_copy(dst_hbm.at[wid], idx_d)
        plsc.subcore_barrier()

        @pl.loop(0, n_batches)
        def _(bi):
            pltpu.async_copy(h_hbm.at[idx_s.at[bi]], rows_a, sem_g).wait()
            pltpu.sync_copy(rows_a, agg_sh.at[idx_d.at[bi]], add=True)
            if with_deg:
                @pl.loop(0, 128, step=16)
                def _(k):
                    plsc.addupdate_scatter(
                        deg_v, [idx_d[bi, pl.ds(k, 16)]],
                        jnp.full((16,), 1.0, _F32))

        plsc.subcore_barrier()
        pltpu.sync_copy(agg_sh.at[pl.ds(r0, rows_ps)],
                        out_hbm.at[c, pl.ds(r0, rows_ps)])
        if with_deg:
            pltpu.sync_copy(deg_v, deg_hbm.at[wid])

    return pl.kernel(body, out_type=tuple(outs) if with_deg else outs[0],
                     mesh=_MESH, scratch_types=scratch,
                     compiler_params=_SC_PARAMS)


def _make_pool():
    rows_ps = _N_T_PAD // _NSUB
    per_w = _CG_PAD // _NW
    t_batches = _E_T_PAD // (_NW * 128)

    outs = (jax.ShapeDtypeStruct((_NCORE, _N_T_PAD, _D), _F32),
            jax.ShapeDtypeStruct((_NW, _N_T_PAD), _F32))
    scratch = [
        pltpu.VMEM((5, 64), _I32),
        pltpu.VMEM((t_batches, 128), _I32),
        pltpu.VMEM((64, _D), _F32),
        pltpu.VMEM_SHARED((_N_T_PAD, _D), _F32),
        pltpu.SemaphoreType.DMA,
        pltpu.VMEM((_N_T_PAD,), _F32),
    ]

    def body(cg_hbm, cols_hbm, tdst_hbm, zeros_hbm, out_hbm, tdeg_hbm,
             idx_p, idx_t, rows, pool_sh, sem, deg_v):
        c = lax.axis_index("c")
        s = lax.axis_index("s")
        wid = c * _NSUB + s
        r0 = s * rows_ps
        pltpu.sync_copy(zeros_hbm.at[pl.ds(r0, rows_ps)],
                        pool_sh.at[pl.ds(r0, rows_ps)])

        @pl.loop(0, _N_T_PAD, step=16)
        def _(i):
            deg_v[pl.ds(i, 16)] = jnp.zeros((16,), _F32)

        pltpu.sync_copy(cols_hbm.at[wid], idx_p)
        pltpu.sync_copy(tdst_hbm.at[wid], idx_t)
        plsc.subcore_barrier()

        base = wid * per_w

        @pl.loop(0, 5)
        def _(bi):
            pltpu.sync_copy(cg_hbm.at[pl.ds(base + bi * 64, 64)], rows)
            pltpu.sync_copy(rows, pool_sh.at[idx_p.at[bi]], add=True)

        @pl.loop(0, t_batches)
        def _(bi):
            @pl.loop(0, 128, step=16)
            def _(k):
                plsc.addupdate_scatter(
                    deg_v, [idx_t[bi, pl.ds(k, 16)]],
                    jnp.full((16,), 1.0, _F32))

        plsc.subcore_barrier()
        pltpu.sync_copy(pool_sh.at[pl.ds(r0, rows_ps)],
                        out_hbm.at[c, pl.ds(r0, rows_ps)])
        pltpu.sync_copy(deg_v, tdeg_hbm.at[wid])

    return pl.kernel(body, out_type=outs, mesh=_MESH, scratch_types=scratch,
                     compiler_params=_SC_PARAMS)


_cell_agg_deg = _make_edge_agg(_N_C_PAD, _E_C_PAD // (_NW * 128), True)
_cell_agg = _make_edge_agg(_N_C_PAD, _E_C_PAD // (_NW * 128), False)
_tissue_agg = _make_edge_agg(_N_T_PAD, _E_T_PAD // (_NW * 128), False)
_pool_scatter = _make_pool()


def _dot(x, w):
    return jax.lax.dot_general(x, w, (((1,), (0,)), ((), ())),
                               preferred_element_type=_F32)


def _norm2_relu(y, prm):
    g1, b1, g2, b2 = prm[1][None], prm[2][None], prm[3][None], prm[4][None]
    mu = jnp.mean(y, axis=0, keepdims=True)
    var = jnp.mean((y - mu) ** 2, axis=0, keepdims=True)
    s1 = jnp.sqrt(var + _EPS)
    v2 = g1 * g1 * var / (var + _EPS)
    s2 = jnp.sqrt(v2 + _EPS)
    return jnp.maximum((y - mu) * (g1 * g2 / (s1 * s2)) + b2, 0.0)


def _combine(p_ref, degp_ref, prm_ref, n):
    y = p_ref[0, :n, :] + p_ref[1, :n, :]
    deg = jnp.maximum(jnp.sum(degp_ref[...], axis=0)[:n], 1.0)
    y = y / deg[:, None] + prm_ref[0][None]
    return _norm2_relu(y, prm_ref)


def _mm_body(x_ref, w_ref, o_ref):
    o_ref[...] = _dot(x_ref[...], w_ref[...])


def _make_post(n):
    def body(p_ref, degp_ref, w_ref, prm_ref, o_ref):
        o_ref[...] = _dot(_combine(p_ref, degp_ref, prm_ref, n), w_ref[...])
    return body


def _make_final(n, out_rows):
    def body(p_ref, degp_ref, prm_ref, o_ref):
        x = _combine(p_ref, degp_ref, prm_ref, n)
        if out_rows > n:
            o_ref[:n] = x
            o_ref[n:] = jnp.zeros((out_rows - n, _D), _F32)
        else:
            o_ref[...] = x
    return body


def _argmax_body(a_ref, o_ref):
    x = a_ref[...]
    m = jnp.max(x, axis=1, keepdims=True)
    ii = lax.broadcasted_iota(_I32, x.shape, 1)
    idx = jnp.min(jnp.where(x == m, ii, jnp.int32(2**30)), axis=1)
    o_ref[...] = idx[:, None].astype(_I32)


def _tg_body(p_ref, tf_ref, w_ref, o_ref):
    tg = p_ref[0, :_N_T, :] + p_ref[1, :_N_T, :] + tf_ref[...]
    o_ref[...] = _dot(tg, w_ref[...])


def _sds(shape, dtype=_F32):
    return jax.ShapeDtypeStruct(shape, dtype)


def kernel(cell_feat, tissue_feat, assignment_mat, Ws, bs, gn_g, gn_b,
           bn_g, bn_b, cell_edge_index, tissue_edge_index):
    nb_c = _E_C_PAD // (_NW * 128)
    nb_t = _E_T_PAD // (_NW * 128)
    zeros_big = jnp.zeros((_N_C_PAD, _D), _F32)

    csrc = jnp.concatenate(
        [cell_edge_index[0], jnp.zeros((_E_C_PAD - _E_C,), _I32)]
    ).reshape(_NW, nb_c, 128)
    cdst = jnp.concatenate(
        [cell_edge_index[1], jnp.full((_E_C_PAD - _E_C,), _N_C, _I32)]
    ).reshape(_NW, nb_c, 128)
    tsrc = jnp.concatenate(
        [tissue_edge_index[0], jnp.zeros((_E_T_PAD - _E_T,), _I32)]
    ).reshape(_NW, nb_t, 128)
    tdst = jnp.concatenate(
        [tissue_edge_index[1], jnp.full((_E_T_PAD - _E_T,), _N_T, _I32)]
    ).reshape(_NW, nb_t, 128)

    prm = [jnp.stack([bs[i], gn_g[i], gn_b[i], bn_g[i], bn_b[i]])
           for i in range(_L)]

    cols2d = pl.pallas_call(
        _argmax_body,
        grid=(5,),
        in_specs=[pl.BlockSpec((2000, _N_T), lambda i: (i, 0))],
        out_specs=pl.BlockSpec((2000, 1), lambda i: (i, 0)),
        out_shape=_sds((_N_C, 1), _I32),
    )(assignment_mat)
    cols = jnp.concatenate(
        [cols2d[:, 0], jnp.zeros((_CG_PAD - _N_C,), _I32)]
    ).reshape(_NW, 5, 64)

    h1 = pl.pallas_call(_mm_body, out_shape=_sds((_N_C, _D)))(
        cell_feat, Ws[0])
    agg1, degp = _cell_agg_deg(h1, csrc, cdst, zeros_big)
    h2 = pl.pallas_call(_make_post(_N_C), out_shape=_sds((_N_C, _D)))(
        agg1, degp, Ws[1], prm[0])
    agg2 = _cell_agg(h2, csrc, cdst, zeros_big)
    cg = pl.pallas_call(_make_final(_N_C, _CG_PAD),
                        out_shape=_sds((_CG_PAD, _D)))(agg2, degp, prm[1])

    poolp, tdegp = _pool_scatter(cg, cols, tdst, zeros_big)
    ht1 = pl.pallas_call(_tg_body, out_shape=_sds((_N_T, _D)))(
        poolp, tissue_feat, Ws[0])

    tagg1 = _tissue_agg(ht1, tsrc, tdst, zeros_big)
    ht2 = pl.pallas_call(_make_post(_N_T), out_shape=_sds((_N_T, _D)))(
        tagg1, tdegp, Ws[1], prm[0])
    tagg2 = _tissue_agg(ht2, tsrc, tdst, zeros_big)
    out = pl.pallas_call(_make_final(_N_T, _N_T),
                         out_shape=_sds((_N_T, _D)))(tagg2, tdegp, prm[1])
    return out


# speedup vs baseline: 1.6116x; 1.0087x over previous
import dataclasses
import functools

import jax
import jax.numpy as jnp
from jax import lax
from jax.experimental import pallas as pl
from jax.experimental.pallas import tpu as pltpu
from jax.experimental.pallas import tpu_sc as plsc

_N_C, _N_T, _D = 10000, 1000, 128
_E_C, _E_T = 320000, 16000
_L = 2
_EPS = 1e-5

_NCORE, _NSUB, _NW = 2, 16, 32
_N_C_PAD = 10112
_N_T_PAD = 1024
_E_C_PAD = 323584
_E_T_PAD = 16384
_CG_PAD = 10240

_MESH = plsc.VectorSubcoreMesh(core_axis_name="c", subcore_axis_name="s")
_F32 = jnp.float32
_I32 = jnp.int32

_SC_PARAMS = pltpu.CompilerParams()
if "needs_layout_passes" in pltpu.CompilerParams.__dataclass_fields__:
    _SC_PARAMS = dataclasses.replace(_SC_PARAMS, needs_layout_passes=False)


def _make_edge_agg(n_pad, n_batches, with_deg):
    rows_ps = n_pad // _NSUB

    outs = [jax.ShapeDtypeStruct((_NCORE, n_pad, _D), _F32)]
    if with_deg:
        outs.append(jax.ShapeDtypeStruct((_NW, n_pad), _F32))
    scratch = [
        pltpu.VMEM((n_batches, 128), _I32),
        pltpu.VMEM((n_batches, 128), _I32),
        pltpu.VMEM((128, _D), _F32),
        pltpu.VMEM_SHARED((n_pad, _D), _F32),
        pltpu.SemaphoreType.DMA,
    ]
    if with_deg:
        scratch += [pltpu.SemaphoreType.DMA,
                    pltpu.VMEM((n_pad,), _F32)]

    def body(h_hbm, src_hbm, dst_hbm, zeros_hbm, *refs):
        if with_deg:
            (out_hbm, deg_hbm, idx_s, idx_d, rows_a, agg_sh, sem_g,
             sem_s, deg_v) = refs
        else:
            (out_hbm, idx_s, idx_d, rows_a, agg_sh, sem_g) = refs
        c = lax.axis_index("c")
        s = lax.axis_index("s")
        wid = c * _NSUB + s
        r0 = s * rows_ps
        pltpu.sync_copy(zeros_hbm.at[pl.ds(r0, rows_ps)],
                        agg_sh.at[pl.ds(r0, rows_ps)])
        if with_deg:
            @pl.loop(0, n_pad, step=16)
            def _(i):
                deg_v[pl.ds(i, 16)] = jnp.zeros((16,), _F32)
        pltpu.sync_copy(src_hbm.at[wid], idx_s)
        pltpu.sync_copy(dst_hbm.at[wid], idx_d)
        plsc.subcore_barrier()

        @pl.loop(0, n_batches)
        def _(bi):
            pltpu.async_copy(h_hbm.at[idx_s.at[bi]], rows_a, sem_g).wait()
            if with_deg:
                scat = pltpu.async_copy(rows_a, agg_sh.at[idx_d.at[bi]],
                                        sem_s, add=True)

                @pl.loop(0, 128, step=16)
                def _(k):
                    plsc.addupdate_scatter(
                        deg_v, [idx_d[bi, pl.ds(k, 16)]],
                        jnp.full((16,), 1.0, _F32))

                scat.wait()
            else:
                pltpu.sync_copy(rows_a, agg_sh.at[idx_d.at[bi]], add=True)

        plsc.subcore_barrier()
        pltpu.sync_copy(agg_sh.at[pl.ds(r0, rows_ps)],
                        out_hbm.at[c, pl.ds(r0, rows_ps)])
        if with_deg:
            pltpu.sync_copy(deg_v, deg_hbm.at[wid])

    return pl.kernel(body, out_type=tuple(outs) if with_deg else outs[0],
                     mesh=_MESH, scratch_types=scratch,
                     compiler_params=_SC_PARAMS)


def _make_pool():
    rows_ps = _N_T_PAD // _NSUB
    per_w = _CG_PAD // _NW
    t_batches = _E_T_PAD // (_NW * 128)

    outs = (jax.ShapeDtypeStruct((_NCORE, _N_T_PAD, _D), _F32),
            jax.ShapeDtypeStruct((_NW, _N_T_PAD), _F32))
    scratch = [
        pltpu.VMEM((5, 64), _I32),
        pltpu.VMEM((t_batches, 128), _I32),
        pltpu.VMEM((64, _D), _F32),
        pltpu.VMEM_SHARED((_N_T_PAD, _D), _F32),
        pltpu.SemaphoreType.DMA,
        pltpu.VMEM((_N_T_PAD,), _F32),
    ]

    def body(cg_hbm, cols_hbm, tdst_hbm, zeros_hbm, out_hbm, tdeg_hbm,
             idx_p, idx_t, rows, pool_sh, sem, deg_v):
        c = lax.axis_index("c")
        s = lax.axis_index("s")
        wid = c * _NSUB + s
        r0 = s * rows_ps
        pltpu.sync_copy(zeros_hbm.at[pl.ds(r0, rows_ps)],
                        pool_sh.at[pl.ds(r0, rows_ps)])

        @pl.loop(0, _N_T_PAD, step=16)
        def _(i):
            deg_v[pl.ds(i, 16)] = jnp.zeros((16,), _F32)

        pltpu.sync_copy(cols_hbm.at[wid], idx_p)
        pltpu.sync_copy(tdst_hbm.at[wid], idx_t)
        plsc.subcore_barrier()

        base = wid * per_w

        @pl.loop(0, 5)
        def _(bi):
            pltpu.sync_copy(cg_hbm.at[pl.ds(base + bi * 64, 64)], rows)
            pltpu.sync_copy(rows, pool_sh.at[idx_p.at[bi]], add=True)

        @pl.loop(0, t_batches)
        def _(bi):
            @pl.loop(0, 128, step=16)
            def _(k):
                plsc.addupdate_scatter(
                    deg_v, [idx_t[bi, pl.ds(k, 16)]],
                    jnp.full((16,), 1.0, _F32))

        plsc.subcore_barrier()
        pltpu.sync_copy(pool_sh.at[pl.ds(r0, rows_ps)],
                        out_hbm.at[c, pl.ds(r0, rows_ps)])
        pltpu.sync_copy(deg_v, tdeg_hbm.at[wid])

    return pl.kernel(body, out_type=outs, mesh=_MESH, scratch_types=scratch,
                     compiler_params=_SC_PARAMS)


_cell_agg_deg = _make_edge_agg(_N_C_PAD, _E_C_PAD // (_NW * 128), True)
_cell_agg = _make_edge_agg(_N_C_PAD, _E_C_PAD // (_NW * 128), False)
_tissue_agg = _make_edge_agg(_N_T_PAD, _E_T_PAD // (_NW * 128), False)
_pool_scatter = _make_pool()


def _dot(x, w):
    return jax.lax.dot_general(x, w, (((1,), (0,)), ((), ())),
                               preferred_element_type=_F32)


def _norm2_relu(y, prm):
    g1, b1, g2, b2 = prm[1][None], prm[2][None], prm[3][None], prm[4][None]
    mu = jnp.mean(y, axis=0, keepdims=True)
    var = jnp.mean((y - mu) ** 2, axis=0, keepdims=True)
    s1 = jnp.sqrt(var + _EPS)
    v2 = g1 * g1 * var / (var + _EPS)
    s2 = jnp.sqrt(v2 + _EPS)
    return jnp.maximum((y - mu) * (g1 * g2 / (s1 * s2)) + b2, 0.0)


def _combine(p_ref, degp_ref, prm_ref, n):
    y = p_ref[0, :n, :] + p_ref[1, :n, :]
    deg = jnp.maximum(jnp.sum(degp_ref[...], axis=0)[:n], 1.0)
    y = y / deg[:, None] + prm_ref[0][None]
    return _norm2_relu(y, prm_ref)


def _mm_body(x_ref, w_ref, o_ref):
    o_ref[...] = _dot(x_ref[...], w_ref[...])


def _make_post(n):
    def body(p_ref, degp_ref, w_ref, prm_ref, o_ref):
        o_ref[...] = _dot(_combine(p_ref, degp_ref, prm_ref, n), w_ref[...])
    return body


def _make_final(n, out_rows):
    def body(p_ref, degp_ref, prm_ref, o_ref):
        x = _combine(p_ref, degp_ref, prm_ref, n)
        if out_rows > n:
            o_ref[:n] = x
            o_ref[n:] = jnp.zeros((out_rows - n, _D), _F32)
        else:
            o_ref[...] = x
    return body


def _argmax_body(a_ref, o_ref):
    x = a_ref[...]
    m = jnp.max(x, axis=1, keepdims=True)
    ii = lax.broadcasted_iota(_I32, x.shape, 1)
    idx = jnp.min(jnp.where(x == m, ii, jnp.int32(2**30)), axis=1)
    o_ref[...] = idx[:, None].astype(_I32)


def _tg_body(p_ref, tf_ref, w_ref, o_ref):
    tg = p_ref[0, :_N_T, :] + p_ref[1, :_N_T, :] + tf_ref[...]
    o_ref[...] = _dot(tg, w_ref[...])


def _sds(shape, dtype=_F32):
    return jax.ShapeDtypeStruct(shape, dtype)


def kernel(cell_feat, tissue_feat, assignment_mat, Ws, bs, gn_g, gn_b,
           bn_g, bn_b, cell_edge_index, tissue_edge_index):
    nb_c = _E_C_PAD // (_NW * 128)
    nb_t = _E_T_PAD // (_NW * 128)
    zeros_big = jnp.zeros((_N_C_PAD, _D), _F32)

    csrc = jnp.concatenate(
        [cell_edge_index[0], jnp.zeros((_E_C_PAD - _E_C,), _I32)]
    ).reshape(_NW, nb_c, 128)
    cdst = jnp.concatenate(
        [cell_edge_index[1], jnp.full((_E_C_PAD - _E_C,), _N_C, _I32)]
    ).reshape(_NW, nb_c, 128)
    tsrc = jnp.concatenate(
        [tissue_edge_index[0], jnp.zeros((_E_T_PAD - _E_T,), _I32)]
    ).reshape(_NW, nb_t, 128)
    tdst = jnp.concatenate(
        [tissue_edge_index[1], jnp.full((_E_T_PAD - _E_T,), _N_T, _I32)]
    ).reshape(_NW, nb_t, 128)

    prm = [jnp.stack([bs[i], gn_g[i], gn_b[i], bn_g[i], bn_b[i]])
           for i in range(_L)]

    cols2d = pl.pallas_call(
        _argmax_body,
        grid=(5,),
        in_specs=[pl.BlockSpec((2000, _N_T), lambda i: (i, 0))],
        out_specs=pl.BlockSpec((2000, 1), lambda i: (i, 0)),
        out_shape=_sds((_N_C, 1), _I32),
    )(assignment_mat)
    cols = jnp.concatenate(
        [cols2d[:, 0], jnp.zeros((_CG_PAD - _N_C,), _I32)]
    ).reshape(_NW, 5, 64)

    h1 = pl.pallas_call(_mm_body, out_shape=_sds((_N_C, _D)))(
        cell_feat, Ws[0])
    agg1, degp = _cell_agg_deg(h1, csrc, cdst, zeros_big)
    h2 = pl.pallas_call(_make_post(_N_C), out_shape=_sds((_N_C, _D)))(
        agg1, degp, Ws[1], prm[0])
    agg2 = _cell_agg(h2, csrc, cdst, zeros_big)
    cg = pl.pallas_call(_make_final(_N_C, _CG_PAD),
                        out_shape=_sds((_CG_PAD, _D)))(agg2, degp, prm[1])

    poolp, tdegp = _pool_scatter(cg, cols, tdst, zeros_big)
    ht1 = pl.pallas_call(_tg_body, out_shape=_sds((_N_T, _D)))(
        poolp, tissue_feat, Ws[0])

    tagg1 = _tissue_agg(ht1, tsrc, tdst, zeros_big)
    ht2 = pl.pallas_call(_make_post(_N_T), out_shape=_sds((_N_T, _D)))(
        tagg1, tdegp, Ws[1], prm[0])
    tagg2 = _tissue_agg(ht2, tsrc, tdst, zeros_big)
    out = pl.pallas_call(_make_final(_N_T, _N_T),
                         out_shape=_sds((_N_T, _D)))(tagg2, tdegp, prm[1])
    return out
